# baseline (speedup 1.0000x reference)
import functools

import jax
import jax.numpy as jnp
from jax import lax
from jax.experimental import pallas as pl
from jax.experimental.pallas import tpu as pltpu
from jax.experimental.pallas import tpu_sc as plsc

VOCAB = 100000
D = 128
C = 100
B = 4096
H = 50

NC = 2
NS = 16
NW = NC * NS
B_PER_W = B // NW
G = 2
ROWS = G * H
ROWS_PAD = 104
N_GROUPS = B_PER_W // G
LANES = 16
D_CHUNKS = D // LANES


def _sum_group(buf, out_v, out_row0):
    for s in range(G):
        row = out_row0 + s
        cur = [buf[s * H, pl.ds(d * LANES, LANES)] for d in range(D_CHUNKS)]
        for d in range(D_CHUNKS):
            out_v[row, pl.ds(d * LANES, LANES)] = cur[d]
        cur = [buf[s * H + 1, pl.ds(d * LANES, LANES)] for d in range(D_CHUNKS)]
        for j in range(1, H):
            nxt = (
                [buf[s * H + j + 1, pl.ds(d * LANES, LANES)] for d in range(D_CHUNKS)]
                if j + 1 < H
                else None
            )
            for d in range(D_CHUNKS):
                plsc.addupdate(out_v.at[row, pl.ds(d * LANES, LANES)], cur[d])
            cur = nxt


NBUF = 4


def _sc_body(table_hbm, idx_hbm, out_hbm, idx_v, bufs, out_v, sems):
    wid = lax.axis_index("s") * NC + lax.axis_index("c")
    base = wid * B_PER_W
    pltpu.sync_copy(idx_hbm.at[wid], idx_v)
    for n in range(NBUF):
        pltpu.async_copy(table_hbm.at[idx_v.at[n]], bufs[n], sems[n])

    def loop_body(i, carry):
        for n in range(NBUF):
            g = NBUF * i + n
            pltpu.make_async_copy(
                table_hbm.at[idx_v.at[g]], bufs[n], sems[n]
            ).wait()
            _sum_group(bufs[n], out_v, 2 * g)

            @pl.when(i < N_GROUPS // NBUF - 1)
            def _():
                pltpu.async_copy(
                    table_hbm.at[idx_v.at[g + NBUF]], bufs[n], sems[n]
                )

        return carry

    lax.fori_loop(0, N_GROUPS // NBUF, loop_body, 0)
    pltpu.sync_copy(out_v, out_hbm.at[pl.ds(base, B_PER_W)])


@functools.partial(jax.jit, static_argnames=())
def _sc_gather_sum(table, idx3):
    mesh = plsc.VectorSubcoreMesh(core_axis_name="c", subcore_axis_name="s")
    return pl.kernel(
        _sc_body,
        out_type=jax.ShapeDtypeStruct((B, D), jnp.float32),
        mesh=mesh,
        scratch_types=[
            pltpu.VMEM((N_GROUPS, ROWS_PAD), jnp.int32),
            [pltpu.VMEM((ROWS_PAD, D), jnp.float32) for _ in range(NBUF)],
            pltpu.VMEM((B_PER_W, D), jnp.float32),
            [pltpu.SemaphoreType.DMA for _ in range(NBUF)],
        ],
    )(table, idx3)


def _head_body(s_ref, w_ref, b_ref, o_ref):
    logits = (
        jnp.dot(s_ref[...], w_ref[...], preferred_element_type=jnp.float32)
        + b_ref[...]
    )
    m = jnp.max(logits, axis=1, keepdims=True)
    lse = jnp.log(jnp.sum(jnp.exp(logits - m), axis=1, keepdims=True)) + m
    o_ref[...] = (logits - lse)[:, :C]


def _head(sum_embed, w_pad_t, b_pad):
    blk = 1024
    return pl.pallas_call(
        _head_body,
        grid=(B // blk,),
        in_specs=[
            pl.BlockSpec((blk, D), lambda i: (i, 0)),
            pl.BlockSpec((D, D), lambda i: (0, 0)),
            pl.BlockSpec((1, D), lambda i: (0, 0)),
        ],
        out_specs=pl.BlockSpec((blk, C), lambda i: (i, 0)),
        out_shape=jax.ShapeDtypeStruct((B, C), jnp.float32),
    )(sum_embed, w_pad_t, b_pad)


def kernel(inputs, table, W, b):
    idx = inputs.astype(jnp.int32).reshape(B // G, ROWS)
    n_pairs = B // G
    pad = (
        jnp.arange(n_pairs * (ROWS_PAD - ROWS), dtype=jnp.int32) % VOCAB
    ).reshape(n_pairs, ROWS_PAD - ROWS)
    idx3 = jnp.concatenate([idx, pad], axis=1).reshape(NW, N_GROUPS, ROWS_PAD)
    sum_embed = _sc_gather_sum(table, idx3)
    w_pad_t = jnp.zeros((D, D), jnp.float32).at[:, :C].set(W.T)
    b_pad = jnp.full((1, D), -1e30, jnp.float32).at[0, :C].set(b)
    return _head(sum_embed, w_pad_t, b_pad)


# speedup vs baseline: 1.2817x; 1.2817x over previous
import functools

import jax
import jax.numpy as jnp
from jax import lax
from jax.experimental import pallas as pl
from jax.experimental.pallas import tpu as pltpu
from jax.experimental.pallas import tpu_sc as plsc

VOCAB = 100000
D = 128
C = 100
B = 4096
H = 50

NC = 2
NS = 16
NW = NC * NS
B_PER_W = B // NW
G = 2
ROWS = G * H
ROWS_PAD = 104
N_GROUPS = B_PER_W // G
LANES = 16
D_CHUNKS = D // LANES


def _sum_group(buf, out_v, out_row0):
    for s in range(G):
        acc = [buf[s * H, pl.ds(d * LANES, LANES)] for d in range(D_CHUNKS)]
        cur = [buf[s * H + 1, pl.ds(d * LANES, LANES)] for d in range(D_CHUNKS)]
        for j in range(1, H):
            nxt = (
                [buf[s * H + j + 1, pl.ds(d * LANES, LANES)] for d in range(D_CHUNKS)]
                if j + 1 < H
                else None
            )
            for d in range(D_CHUNKS):
                acc[d] = acc[d] + cur[d]
            cur = nxt
        for d in range(D_CHUNKS):
            out_v[out_row0 + s, pl.ds(d * LANES, LANES)] = acc[d]


NBUF = 4


def _sc_body(table_hbm, idx_hbm, out_hbm, idx_v, bufs, out_v, sems):
    wid = lax.axis_index("s") * NC + lax.axis_index("c")
    base = wid * B_PER_W
    pltpu.sync_copy(idx_hbm.at[wid], idx_v)
    for n in range(NBUF):
        pltpu.async_copy(table_hbm.at[idx_v.at[n]], bufs[n], sems[n])

    def loop_body(i, carry):
        for n in range(NBUF):
            g = NBUF * i + n
            pltpu.make_async_copy(
                table_hbm.at[idx_v.at[g]], bufs[n], sems[n]
            ).wait()
            _sum_group(bufs[n], out_v, 2 * g)

            @pl.when(i < N_GROUPS // NBUF - 1)
            def _():
                pltpu.async_copy(
                    table_hbm.at[idx_v.at[g + NBUF]], bufs[n], sems[n]
                )

        return carry

    lax.fori_loop(0, N_GROUPS // NBUF, loop_body, 0)
    pltpu.sync_copy(out_v, out_hbm.at[pl.ds(base, B_PER_W)])


@functools.partial(jax.jit, static_argnames=())
def _sc_gather_sum(table, idx3):
    mesh = plsc.VectorSubcoreMesh(core_axis_name="c", subcore_axis_name="s")
    return pl.kernel(
        _sc_body,
        out_type=jax.ShapeDtypeStruct((B, D), jnp.float32),
        mesh=mesh,
        scratch_types=[
            pltpu.VMEM((N_GROUPS, ROWS_PAD), jnp.int32),
            [pltpu.VMEM((ROWS_PAD, D), jnp.float32) for _ in range(NBUF)],
            pltpu.VMEM((B_PER_W, D), jnp.float32),
            [pltpu.SemaphoreType.DMA for _ in range(NBUF)],
        ],
    )(table, idx3)


def _head_body(s_ref, w_ref, b_ref, o_ref):
    logits = (
        jnp.dot(s_ref[...], w_ref[...], preferred_element_type=jnp.float32)
        + b_ref[...]
    )
    m = jnp.max(logits, axis=1, keepdims=True)
    lse = jnp.log(jnp.sum(jnp.exp(logits - m), axis=1, keepdims=True)) + m
    o_ref[...] = (logits - lse)[:, :C]


def _head(sum_embed, w_pad_t, b_pad):
    blk = 1024
    return pl.pallas_call(
        _head_body,
        grid=(B // blk,),
        in_specs=[
            pl.BlockSpec((blk, D), lambda i: (i, 0)),
            pl.BlockSpec((D, D), lambda i: (0, 0)),
            pl.BlockSpec((1, D), lambda i: (0, 0)),
        ],
        out_specs=pl.BlockSpec((blk, C), lambda i: (i, 0)),
        out_shape=jax.ShapeDtypeStruct((B, C), jnp.float32),
    )(sum_embed, w_pad_t, b_pad)


def kernel(inputs, table, W, b):
    idx = inputs.astype(jnp.int32).reshape(B // G, ROWS)
    n_pairs = B // G
    pad = (
        jnp.arange(n_pairs * (ROWS_PAD - ROWS), dtype=jnp.int32) % VOCAB
    ).reshape(n_pairs, ROWS_PAD - ROWS)
    idx3 = jnp.concatenate([idx, pad], axis=1).reshape(NW, N_GROUPS, ROWS_PAD)
    sum_embed = _sc_gather_sum(table, idx3)
    w_pad_t = jnp.zeros((D, D), jnp.float32).at[:, :C].set(W.T)
    b_pad = jnp.full((1, D), -1e30, jnp.float32).at[0, :C].set(b)
    return _head(sum_embed, w_pad_t, b_pad)
